# baseline (speedup 1.0000x reference)
import jax
import jax.numpy as jnp
from jax.experimental import pallas as pl
from jax.experimental.pallas import tpu as pltpu

_G = 2
_ITERS = 30
_B = 32
_N = 512
_D = 512
_P = _B * _N
_CH = 2048
_NCH = _P // _CH


def _kmeans_body(x_hbm, c0_ref, out_ref,
                 xnt_ref, xbt_ref, buf_ref, sems):
    f32 = jnp.float32
    bf16 = jnp.bfloat16

    def copy_in(i, slot):
        return pltpu.make_async_copy(
            x_hbm.at[pl.ds(i * _CH, _CH), :], buf_ref.at[slot], sems.at[slot])

    copy_in(0, 0).start()

    def chunk_step(i, carry):
        slot = jax.lax.rem(i, 2)

        @pl.when(i + 1 < _NCH)
        def _start_next():
            copy_in(i + 1, 1 - slot).start()

        copy_in(i, slot).wait()
        xc = buf_ref[slot]
        nc = jnp.sqrt(jnp.sum(xc * xc, axis=1, keepdims=True))
        xbt_ref[:, pl.ds(i * _CH, _CH)] = xc.astype(bf16).T
        xnbc = (xc / (nc + 1e-8)).astype(bf16)
        xnt_ref[:, pl.ds(i * _CH, _CH)] = xnbc.T
        return carry

    jax.lax.fori_loop(0, _NCH, chunk_step, 0)

    def assign_onehot(centers8):
        norm = jnp.sqrt(jnp.sum(centers8 * centers8, axis=1, keepdims=True))
        cn = (centers8 / (norm + 1e-8)).astype(bf16)
        simT = jax.lax.dot_general(
            cn, xnt_ref[...],
            dimension_numbers=(((1,), (0,)), ((), ())),
            preferred_element_type=f32)
        m1 = (simT[1:2, :] > simT[0:1, :]).astype(bf16)
        onehotT = jnp.concatenate(
            [1.0 - m1, m1, jnp.zeros((6, _P), bf16)], axis=0)
        return onehotT

    def update_centers(onehotT):
        sumsT = jax.lax.dot_general(
            xbt_ref[...], onehotT,
            dimension_numbers=(((1,), (1,)), ((), ())),
            preferred_element_type=f32)
        sums = sumsT.T
        cnt1 = jnp.sum(onehotT[1:2, :].astype(f32))
        row = jax.lax.broadcasted_iota(jnp.int32, (8, 1), 0)
        counts8 = jnp.where(row == 0, _P - cnt1,
                            jnp.where(row == 1, cnt1, 0.0))
        return sums / jnp.maximum(counts8, 1.0)

    centers8 = jnp.concatenate(
        [c0_ref[...], jnp.zeros((8 - _G, _D), f32)], axis=0)

    def body(_, c8):
        return update_centers(assign_onehot(c8))

    centers8 = jax.lax.fori_loop(0, _ITERS - 1, body, centers8)
    onehotT = assign_onehot(centers8)

    for b in range(_B):
        s = b * _N
        agg = jax.lax.dot_general(
            onehotT[:, s:s + _N], xbt_ref[:, s:s + _N],
            dimension_numbers=(((1,), (1,)), ((), ())),
            preferred_element_type=f32)
        out_ref[b, :, :] = agg[:_G, :]


def kernel(person_feats_thisbatch_proj):
    B, N, d = person_feats_thisbatch_proj.shape
    X = person_feats_thisbatch_proj.reshape(B * N, d)
    init_idx = jax.random.choice(
        jax.random.key(42), B * N, shape=(_G,), replace=False)
    c0 = X[init_idx]
    return pl.pallas_call(
        _kmeans_body,
        in_specs=[
            pl.BlockSpec(memory_space=pltpu.MemorySpace.HBM),
            pl.BlockSpec(memory_space=pltpu.MemorySpace.VMEM),
        ],
        out_specs=pl.BlockSpec(memory_space=pltpu.MemorySpace.VMEM),
        scratch_shapes=[
            pltpu.VMEM((_D, _P), jnp.bfloat16),
            pltpu.VMEM((_D, _P), jnp.bfloat16),
            pltpu.VMEM((2, _CH, _D), jnp.float32),
            pltpu.SemaphoreType.DMA((2,)),
        ],
        out_shape=jax.ShapeDtypeStruct((B, _G, d), jnp.float32),
    )(X, c0)


# speedup vs baseline: 1.3545x; 1.3545x over previous
import jax
import jax.numpy as jnp
from jax.experimental import pallas as pl
from jax.experimental.pallas import tpu as pltpu

_G = 2
_ITERS = 30
_B = 32
_N = 512
_D = 512
_P = _B * _N
_CH = 2048
_NCH = _P // _CH


def _kmeans_body(x_hbm, c0_ref, out_ref,
                 xnt_ref, xb_ref, buf_ref, sems):
    f32 = jnp.float32
    bf16 = jnp.bfloat16

    def copy_in(i, slot):
        return pltpu.make_async_copy(
            x_hbm.at[pl.ds(i * _CH, _CH), :], buf_ref.at[slot], sems.at[slot])

    copy_in(0, 0).start()

    def chunk_step(i, carry):
        slot = jax.lax.rem(i, 2)

        @pl.when(i + 1 < _NCH)
        def _start_next():
            copy_in(i + 1, 1 - slot).start()

        copy_in(i, slot).wait()
        xc = buf_ref[slot]
        nc = jnp.sqrt(jnp.sum(xc * xc, axis=1, keepdims=True))
        xb_ref[pl.ds(i * _CH, _CH), :] = xc.astype(bf16)
        xnbc = (xc / (nc + 1e-8)).astype(bf16)
        xnt_ref[:, pl.ds(i * _CH, _CH)] = xnbc.T
        return carry

    jax.lax.fori_loop(0, _NCH, chunk_step, 0)

    def assign_onehot(centers8):
        norm = jnp.sqrt(jnp.sum(centers8 * centers8, axis=1, keepdims=True))
        cn = (centers8 / (norm + 1e-8)).astype(bf16)
        simT = jax.lax.dot_general(
            cn, xnt_ref[...],
            dimension_numbers=(((1,), (0,)), ((), ())),
            preferred_element_type=f32)
        m1 = (simT[1:2, :] > simT[0:1, :]).astype(bf16)
        onehotT = jnp.concatenate(
            [1.0 - m1, m1, jnp.zeros((6, _P), bf16)], axis=0)
        return onehotT

    def update_centers(onehotT):
        sums = jax.lax.dot_general(
            onehotT, xb_ref[...],
            dimension_numbers=(((1,), (0,)), ((), ())),
            preferred_element_type=f32)
        cnt1 = jnp.sum(onehotT[1:2, :].astype(f32))
        row = jax.lax.broadcasted_iota(jnp.int32, (8, 1), 0)
        counts8 = jnp.where(row == 0, _P - cnt1,
                            jnp.where(row == 1, cnt1, 0.0))
        return sums / jnp.maximum(counts8, 1.0)

    centers8 = jnp.concatenate(
        [c0_ref[...], jnp.zeros((8 - _G, _D), f32)], axis=0)

    def body(_, c8):
        return update_centers(assign_onehot(c8))

    centers8 = jax.lax.fori_loop(0, _ITERS - 1, body, centers8)
    onehotT = assign_onehot(centers8)

    for b in range(_B):
        s = b * _N
        agg = jax.lax.dot_general(
            onehotT[:, s:s + _N], xb_ref[s:s + _N, :],
            dimension_numbers=(((1,), (0,)), ((), ())),
            preferred_element_type=f32)
        out_ref[b, :, :] = agg[:_G, :]


def kernel(person_feats_thisbatch_proj):
    B, N, d = person_feats_thisbatch_proj.shape
    X = person_feats_thisbatch_proj.reshape(B * N, d)
    init_idx = jax.random.choice(
        jax.random.key(42), B * N, shape=(_G,), replace=False)
    c0 = X[init_idx]
    return pl.pallas_call(
        _kmeans_body,
        in_specs=[
            pl.BlockSpec(memory_space=pltpu.MemorySpace.HBM),
            pl.BlockSpec(memory_space=pltpu.MemorySpace.VMEM),
        ],
        out_specs=pl.BlockSpec(memory_space=pltpu.MemorySpace.VMEM),
        scratch_shapes=[
            pltpu.VMEM((_D, _P), jnp.bfloat16),
            pltpu.VMEM((_P, _D), jnp.bfloat16),
            pltpu.VMEM((2, _CH, _D), jnp.float32),
            pltpu.SemaphoreType.DMA((2,)),
        ],
        out_shape=jax.ShapeDtypeStruct((B, _G, d), jnp.float32),
    )(X, c0)
